# baseline (speedup 1.0000x reference)
import functools

import jax
import jax.numpy as jnp
from jax import lax
from jax.experimental import pallas as pl
from jax.experimental.pallas import tpu as pltpu
from jax.experimental.pallas import tpu_sc as plsc

_EPS = 1e-5
_S_BLK = 2048


def _ln_body(x_ref, pos_ref, g_ref, b_ref, o_ref):
    x = x_ref[...]
    p = pos_ref[...]
    e = x + p[None, :, :]
    mean = jnp.mean(e, axis=-1, keepdims=True)
    c = e - mean
    var = jnp.mean(c * c, axis=-1, keepdims=True)
    o_ref[...] = c * jax.lax.rsqrt(var + _EPS) * g_ref[...] + b_ref[...]


def _rsqrt_newton(v):
    i = plsc.bitcast(v, jnp.int32)
    i = jnp.int32(0x5F3759DF) - (i >> 1)
    y = plsc.bitcast(i, jnp.float32)
    for _ in range(3):
        y = y * (1.5 - 0.5 * v * y * y)
    return y


def _sc_layernorm(input_feat, pos_table, ln_gamma, ln_beta):
    bsz, seq, d = input_feat.shape
    info = plsc.get_sparse_core_info()
    nw = info.num_cores * info.num_subcores
    lanes = info.num_lanes
    span = seq // nw
    r = lanes
    nchunk = span // r
    nvec = d // lanes
    nstep = nchunk * bsz
    mesh = plsc.VectorSubcoreMesh(core_axis_name="c", subcore_axis_name="s")

    @functools.partial(
        pl.kernel,
        mesh=mesh,
        out_type=jax.ShapeDtypeStruct((bsz, seq, d), jnp.float32),
        scratch_types=[
            pltpu.VMEM((2, r, d), jnp.float32),
            pltpu.VMEM((2, r, d), jnp.float32),
            pltpu.VMEM((r, d), jnp.float32),
            pltpu.VMEM((2, r, d), jnp.float32),
            pltpu.VMEM((d,), jnp.float32),
            pltpu.VMEM((d,), jnp.float32),
            pltpu.SemaphoreType.DMA((2,)),
            pltpu.SemaphoreType.DMA((2,)),
            pltpu.SemaphoreType.DMA((2,)),
        ],
        compiler_params=pltpu.CompilerParams(
            use_tc_tiling_on_sc=False, needs_layout_passes=False
        ),
    )
    def k(x_hbm, pos_hbm, g_hbm, b_hbm, out_hbm,
          xbuf, posbuf, ebuf, obuf, gbuf, bbuf, sem_x, sem_p, sem_o):
        wid = lax.axis_index("s") * info.num_cores + lax.axis_index("c")
        s0 = wid * span
        pltpu.sync_copy(g_hbm, gbuf)
        pltpu.sync_copy(b_hbm, bbuf)

        def x_src(t):
            return x_hbm.at[t % bsz, pl.ds(s0 + (t // bsz) * r, r)]

        pltpu.async_copy(x_src(0), xbuf.at[0], sem_x.at[0])
        pltpu.async_copy(pos_hbm.at[pl.ds(s0, r)], posbuf.at[0], sem_p.at[0])

        def step(t, _):
            slot = t % 2
            ci = t // bsz
            pslot = ci % 2

            @pl.when(t % bsz == 0)
            def _chunk_boundary():
                pltpu.make_async_copy(
                    pos_hbm.at[pl.ds(s0, r)], posbuf.at[pslot], sem_p.at[pslot]
                ).wait()

                @pl.when(ci + 1 < nchunk)
                def _():
                    pltpu.async_copy(
                        pos_hbm.at[pl.ds(s0 + (ci + 1) * r, r)],
                        posbuf.at[1 - pslot],
                        sem_p.at[1 - pslot],
                    )

            pltpu.make_async_copy(x_src(t), xbuf.at[slot], sem_x.at[slot]).wait()

            @pl.when(t + 1 < nstep)
            def _():
                pltpu.async_copy(x_src(t + 1), xbuf.at[1 - slot], sem_x.at[1 - slot])

            @pl.when(t >= 2)
            def _():
                pltpu.make_async_copy(
                    x_src(t), obuf.at[slot], sem_o.at[slot]
                ).wait()

            def row_body(rw, _):
                xr = xbuf.at[slot, rw]
                pr = posbuf.at[pslot, rw]
                er = ebuf.at[rw]
                orow = obuf.at[slot, rw]
                zero = jnp.zeros((lanes,), jnp.float32)
                s = [zero] * 4
                q = [zero] * 4
                for v in range(nvec):
                    off = v * lanes
                    e = xr[pl.ds(off, lanes)] + pr[pl.ds(off, lanes)]
                    er[pl.ds(off, lanes)] = e
                    s[v % 4] = s[v % 4] + e
                    q[v % 4] = q[v % 4] + e * e
                mean = jnp.sum((s[0] + s[1]) + (s[2] + s[3])) * (1.0 / d)
                var = jnp.sum((q[0] + q[1]) + (q[2] + q[3])) * (1.0 / d) - mean * mean
                mv = jnp.full((lanes,), mean)
                rstd = _rsqrt_newton(jnp.full((lanes,), var + _EPS))
                for v in range(nvec):
                    off = v * lanes
                    e = er[pl.ds(off, lanes)]
                    orow[pl.ds(off, lanes)] = (
                        (e - mv) * rstd * gbuf[pl.ds(off, lanes)]
                        + bbuf[pl.ds(off, lanes)]
                    )
                return 0

            lax.fori_loop(0, r, row_body, 0)

            pltpu.async_copy(
                obuf.at[slot],
                out_hbm.at[t % bsz, pl.ds(s0 + ci * r, r)],
                sem_o.at[slot],
            )
            return 0

        lax.fori_loop(0, nstep, step, 0)
        for slot in range(2):
            pltpu.make_async_copy(x_src(0), obuf.at[slot], sem_o.at[slot]).wait()

    return k(input_feat, pos_table, ln_gamma, ln_beta)


def _tc_layernorm(input_feat, pos_table, ln_gamma, ln_beta):
    bsz, seq, d = input_feat.shape
    n_s = seq // _S_BLK
    grid = (n_s, bsz)
    return pl.pallas_call(
        _ln_body,
        grid=grid,
        in_specs=[
            pl.BlockSpec((1, _S_BLK, d), lambda i, j: (j, i, 0)),
            pl.BlockSpec((_S_BLK, d), lambda i, j: (i, 0)),
            pl.BlockSpec((d,), lambda i, j: (0,)),
            pl.BlockSpec((d,), lambda i, j: (0,)),
        ],
        out_specs=pl.BlockSpec((1, _S_BLK, d), lambda i, j: (j, i, 0)),
        out_shape=jax.ShapeDtypeStruct((bsz, seq, d), input_feat.dtype),
        compiler_params=pltpu.CompilerParams(
            dimension_semantics=("arbitrary", "arbitrary"),
            vmem_limit_bytes=128 * 1024 * 1024,
        ),
    )(input_feat, pos_table, ln_gamma, ln_beta)


def kernel(input_feat, pos_table, ln_gamma, ln_beta):
    return _sc_layernorm(input_feat, pos_table, ln_gamma, ln_beta)


# speedup vs baseline: 11.7560x; 11.7560x over previous
import functools

import jax
import jax.numpy as jnp
from jax import lax
from jax.experimental import pallas as pl
from jax.experimental.pallas import tpu as pltpu
from jax.experimental.pallas import tpu_sc as plsc

_EPS = 1e-5
_S_BLK = 2048


def _ln_body(x_ref, pos_ref, g_ref, b_ref, o_ref):
    x = x_ref[...]
    p = pos_ref[...]
    e = x + p[None, :, :]
    mean = jnp.mean(e, axis=-1, keepdims=True)
    c = e - mean
    var = jnp.mean(c * c, axis=-1, keepdims=True)
    o_ref[...] = c * jax.lax.rsqrt(var + _EPS) * g_ref[...] + b_ref[...]


def _rsqrt_newton(v):
    i = plsc.bitcast(v, jnp.int32)
    i = jnp.int32(0x5F3759DF) - (i >> 1)
    y = plsc.bitcast(i, jnp.float32)
    for _ in range(3):
        y = y * (1.5 - 0.5 * v * y * y)
    return y


def _sc_layernorm(input_feat, pos_table, ln_gamma, ln_beta):
    bsz, seq, d = input_feat.shape
    info = plsc.get_sparse_core_info()
    nw = info.num_cores * info.num_subcores
    lanes = info.num_lanes
    span = seq // nw
    r = lanes
    nchunk = span // r
    nvec = d // lanes
    nstep = nchunk * bsz
    mesh = plsc.VectorSubcoreMesh(core_axis_name="c", subcore_axis_name="s")

    @functools.partial(
        pl.kernel,
        mesh=mesh,
        out_type=jax.ShapeDtypeStruct((bsz, seq, d), jnp.float32),
        scratch_types=[
            pltpu.VMEM((2, r, d), jnp.float32),
            pltpu.VMEM((2, r, d), jnp.float32),
            pltpu.VMEM((r, d), jnp.float32),
            pltpu.VMEM((2, r, d), jnp.float32),
            pltpu.VMEM((d,), jnp.float32),
            pltpu.VMEM((d,), jnp.float32),
            pltpu.SemaphoreType.DMA((2,)),
            pltpu.SemaphoreType.DMA((2,)),
            pltpu.SemaphoreType.DMA((2,)),
        ],
        compiler_params=pltpu.CompilerParams(
            use_tc_tiling_on_sc=False, needs_layout_passes=False
        ),
    )
    def k(x_hbm, pos_hbm, g_hbm, b_hbm, out_hbm,
          xbuf, posbuf, ebuf, obuf, gbuf, bbuf, sem_x, sem_p, sem_o):
        wid = lax.axis_index("s") * info.num_cores + lax.axis_index("c")
        s0 = wid * span
        pltpu.sync_copy(g_hbm, gbuf)
        pltpu.sync_copy(b_hbm, bbuf)

        def x_src(t):
            return x_hbm.at[t % bsz, pl.ds(s0 + (t // bsz) * r, r)]

        pltpu.async_copy(x_src(0), xbuf.at[0], sem_x.at[0])
        pltpu.async_copy(pos_hbm.at[pl.ds(s0, r)], posbuf.at[0], sem_p.at[0])

        def step(t, _):
            slot = t % 2
            ci = t // bsz
            pslot = ci % 2

            @pl.when(t % bsz == 0)
            def _chunk_boundary():
                pltpu.make_async_copy(
                    pos_hbm.at[pl.ds(s0, r)], posbuf.at[pslot], sem_p.at[pslot]
                ).wait()

                @pl.when(ci + 1 < nchunk)
                def _():
                    pltpu.async_copy(
                        pos_hbm.at[pl.ds(s0 + (ci + 1) * r, r)],
                        posbuf.at[1 - pslot],
                        sem_p.at[1 - pslot],
                    )

            pltpu.make_async_copy(x_src(t), xbuf.at[slot], sem_x.at[slot]).wait()

            @pl.when(t + 1 < nstep)
            def _():
                pltpu.async_copy(x_src(t + 1), xbuf.at[1 - slot], sem_x.at[1 - slot])

            @pl.when(t >= 2)
            def _():
                pltpu.make_async_copy(
                    x_src(t), obuf.at[slot], sem_o.at[slot]
                ).wait()

            def row_body(rw, _):
                xr = xbuf.at[slot, rw]
                pr = posbuf.at[pslot, rw]
                er = ebuf.at[rw]
                orow = obuf.at[slot, rw]
                zero = jnp.zeros((lanes,), jnp.float32)
                s = [zero] * 4
                q = [zero] * 4
                for v in range(nvec):
                    off = v * lanes
                    e = xr[pl.ds(off, lanes)] + pr[pl.ds(off, lanes)]
                    er[pl.ds(off, lanes)] = e
                    s[v % 4] = s[v % 4] + e
                    q[v % 4] = q[v % 4] + e * e
                mean = jnp.sum((s[0] + s[1]) + (s[2] + s[3])) * (1.0 / d)
                var = jnp.sum((q[0] + q[1]) + (q[2] + q[3])) * (1.0 / d) - mean * mean
                mv = jnp.full((lanes,), mean)
                rstd = _rsqrt_newton(jnp.full((lanes,), var + _EPS))
                for v in range(nvec):
                    off = v * lanes
                    e = er[pl.ds(off, lanes)]
                    orow[pl.ds(off, lanes)] = (
                        (e - mv) * rstd * gbuf[pl.ds(off, lanes)]
                        + bbuf[pl.ds(off, lanes)]
                    )
                return 0

            lax.fori_loop(0, r, row_body, 0)

            pltpu.async_copy(
                obuf.at[slot],
                out_hbm.at[t % bsz, pl.ds(s0 + ci * r, r)],
                sem_o.at[slot],
            )
            return 0

        lax.fori_loop(0, nstep, step, 0)
        for slot in range(2):
            pltpu.make_async_copy(x_src(0), obuf.at[slot], sem_o.at[slot]).wait()

    return k(input_feat, pos_table, ln_gamma, ln_beta)


def _tc_layernorm(input_feat, pos_table, ln_gamma, ln_beta):
    bsz, seq, d = input_feat.shape
    n_s = seq // _S_BLK
    grid = (n_s, bsz)
    return pl.pallas_call(
        _ln_body,
        grid=grid,
        in_specs=[
            pl.BlockSpec((1, _S_BLK, d), lambda i, j: (j, i, 0)),
            pl.BlockSpec((_S_BLK, d), lambda i, j: (i, 0)),
            pl.BlockSpec((d,), lambda i, j: (0,)),
            pl.BlockSpec((d,), lambda i, j: (0,)),
        ],
        out_specs=pl.BlockSpec((1, _S_BLK, d), lambda i, j: (j, i, 0)),
        out_shape=jax.ShapeDtypeStruct((bsz, seq, d), input_feat.dtype),
        compiler_params=pltpu.CompilerParams(
            dimension_semantics=("arbitrary", "arbitrary"),
            vmem_limit_bytes=128 * 1024 * 1024,
        ),
    )(input_feat, pos_table, ln_gamma, ln_beta)


def kernel(input_feat, pos_table, ln_gamma, ln_beta):
    return _tc_layernorm(input_feat, pos_table, ln_gamma, ln_beta)


# speedup vs baseline: 11.7574x; 1.0001x over previous
import functools

import jax
import jax.numpy as jnp
from jax import lax
from jax.experimental import pallas as pl
from jax.experimental.pallas import tpu as pltpu
from jax.experimental.pallas import tpu_sc as plsc

_EPS = 1e-5
_S_BLK = 2048


def _ln_body(x_ref, pos_ref, g_ref, b_ref, o_ref):
    x = x_ref[...]
    p = pos_ref[...]
    e = x + p[None, :, :]
    mean = jnp.mean(e, axis=-1, keepdims=True)
    c = e - mean
    var = jnp.mean(c * c, axis=-1, keepdims=True)
    o_ref[...] = c * jax.lax.rsqrt(var + _EPS) * g_ref[...] + b_ref[...]


def _rsqrt_newton(v):
    i = plsc.bitcast(v, jnp.int32)
    i = jnp.int32(0x5F3759DF) - (i >> 1)
    y = plsc.bitcast(i, jnp.float32)
    for _ in range(3):
        y = y * (1.5 - 0.5 * v * y * y)
    return y


def _sc_layernorm(input_feat, pos_table, ln_gamma, ln_beta):
    bsz, seq, d = input_feat.shape
    info = plsc.get_sparse_core_info()
    nw = info.num_cores * info.num_subcores
    lanes = info.num_lanes
    span = seq // nw
    r = lanes
    nchunk = span // r
    nvec = d // lanes
    nstep = nchunk * bsz
    mesh = plsc.VectorSubcoreMesh(core_axis_name="c", subcore_axis_name="s")

    @functools.partial(
        pl.kernel,
        mesh=mesh,
        out_type=jax.ShapeDtypeStruct((bsz, seq, d), jnp.float32),
        scratch_types=[
            pltpu.VMEM((2, r, d), jnp.float32),
            pltpu.VMEM((2, r, d), jnp.float32),
            pltpu.VMEM((r, d), jnp.float32),
            pltpu.VMEM((2, r, d), jnp.float32),
            pltpu.VMEM((d,), jnp.float32),
            pltpu.VMEM((d,), jnp.float32),
            pltpu.SemaphoreType.DMA((2,)),
            pltpu.SemaphoreType.DMA((2,)),
            pltpu.SemaphoreType.DMA((2,)),
        ],
        compiler_params=pltpu.CompilerParams(
            use_tc_tiling_on_sc=False, needs_layout_passes=False
        ),
    )
    def k(x_hbm, pos_hbm, g_hbm, b_hbm, out_hbm,
          xbuf, posbuf, ebuf, obuf, gbuf, bbuf, sem_x, sem_p, sem_o):
        wid = lax.axis_index("s") * info.num_cores + lax.axis_index("c")
        s0 = wid * span
        pltpu.sync_copy(g_hbm, gbuf)
        pltpu.sync_copy(b_hbm, bbuf)

        def x_src(t):
            return x_hbm.at[t % bsz, pl.ds(s0 + (t // bsz) * r, r)]

        pltpu.async_copy(x_src(0), xbuf.at[0], sem_x.at[0])
        pltpu.async_copy(pos_hbm.at[pl.ds(s0, r)], posbuf.at[0], sem_p.at[0])

        def step(t, _):
            slot = t % 2
            ci = t // bsz
            pslot = ci % 2

            @pl.when(t % bsz == 0)
            def _chunk_boundary():
                pltpu.make_async_copy(
                    pos_hbm.at[pl.ds(s0, r)], posbuf.at[pslot], sem_p.at[pslot]
                ).wait()

                @pl.when(ci + 1 < nchunk)
                def _():
                    pltpu.async_copy(
                        pos_hbm.at[pl.ds(s0 + (ci + 1) * r, r)],
                        posbuf.at[1 - pslot],
                        sem_p.at[1 - pslot],
                    )

            pltpu.make_async_copy(x_src(t), xbuf.at[slot], sem_x.at[slot]).wait()

            @pl.when(t + 1 < nstep)
            def _():
                pltpu.async_copy(x_src(t + 1), xbuf.at[1 - slot], sem_x.at[1 - slot])

            @pl.when(t >= 2)
            def _():
                pltpu.make_async_copy(
                    x_src(t), obuf.at[slot], sem_o.at[slot]
                ).wait()

            def row_body(rw, _):
                xr = xbuf.at[slot, rw]
                pr = posbuf.at[pslot, rw]
                er = ebuf.at[rw]
                orow = obuf.at[slot, rw]
                zero = jnp.zeros((lanes,), jnp.float32)
                s = [zero] * 4
                q = [zero] * 4
                for v in range(nvec):
                    off = v * lanes
                    e = xr[pl.ds(off, lanes)] + pr[pl.ds(off, lanes)]
                    er[pl.ds(off, lanes)] = e
                    s[v % 4] = s[v % 4] + e
                    q[v % 4] = q[v % 4] + e * e
                mean = jnp.sum((s[0] + s[1]) + (s[2] + s[3])) * (1.0 / d)
                var = jnp.sum((q[0] + q[1]) + (q[2] + q[3])) * (1.0 / d) - mean * mean
                mv = jnp.full((lanes,), mean)
                rstd = _rsqrt_newton(jnp.full((lanes,), var + _EPS))
                for v in range(nvec):
                    off = v * lanes
                    e = er[pl.ds(off, lanes)]
                    orow[pl.ds(off, lanes)] = (
                        (e - mv) * rstd * gbuf[pl.ds(off, lanes)]
                        + bbuf[pl.ds(off, lanes)]
                    )
                return 0

            lax.fori_loop(0, r, row_body, 0)

            pltpu.async_copy(
                obuf.at[slot],
                out_hbm.at[t % bsz, pl.ds(s0 + ci * r, r)],
                sem_o.at[slot],
            )
            return 0

        lax.fori_loop(0, nstep, step, 0)
        for slot in range(2):
            pltpu.make_async_copy(x_src(0), obuf.at[slot], sem_o.at[slot]).wait()

    return k(input_feat, pos_table, ln_gamma, ln_beta)


def _tc_layernorm(input_feat, pos_table, ln_gamma, ln_beta):
    bsz, seq, d = input_feat.shape
    n_s = seq // _S_BLK
    grid = (n_s, bsz)
    return pl.pallas_call(
        _ln_body,
        grid=grid,
        in_specs=[
            pl.BlockSpec((1, _S_BLK, d), lambda i, j: (j, i, 0)),
            pl.BlockSpec((_S_BLK, d), lambda i, j: (i, 0)),
            pl.BlockSpec((d,), lambda i, j: (0,)),
            pl.BlockSpec((d,), lambda i, j: (0,)),
        ],
        out_specs=pl.BlockSpec((1, _S_BLK, d), lambda i, j: (j, i, 0)),
        out_shape=jax.ShapeDtypeStruct((bsz, seq, d), input_feat.dtype),
        compiler_params=pltpu.CompilerParams(
            dimension_semantics=("arbitrary", "arbitrary"),
        ),
    )(input_feat, pos_table, ln_gamma, ln_beta)


def kernel(input_feat, pos_table, ln_gamma, ln_beta):
    return _tc_layernorm(input_feat, pos_table, ln_gamma, ln_beta)
